# Initial kernel scaffold; baseline (speedup 1.0000x reference)
import jax
import jax.numpy as jnp
from jax.experimental import pallas as pl


def kernel(x, edge_index, edge_attr, agg_W, agg_b, e_W1, e_b1, e_W2, e_b2, u_W1, u_b1, u_W2, u_b2, gamma, beta, res_W):
    raise NotImplementedError("write your pallas kernel here")


# speedup vs baseline: 2.1724x; 2.1724x over previous
import functools

import jax
import jax.numpy as jnp
from jax import lax
from jax.experimental import pallas as pl
from jax.experimental.pallas import tpu as pltpu
from jax.experimental.pallas import tpu_sc as plsc

N = 10000
E = 320000
D = 128
DE = 4
NC = 2
NS = 16
L = 16
NW = NC * NS
EPW = E // NW
C = 80
NCHUNK = EPW // C
RPT = N // NS
CNT_W = 16

BE = 8000
BN = 2000

_HI = lax.Precision.HIGHEST


def _silu(v):
    return v * jax.nn.sigmoid(v)


def _dot(a, b):
    return jnp.dot(a, b, precision=_HI, preferred_element_type=jnp.float32)


def _gate_body(ea_ref, w1_ref, b1_ref, w2_ref, b2_ref, o_ref):
    eh = _silu(_dot(ea_ref[...], w1_ref[...]) + b1_ref[...])
    o_ref[...] = jax.nn.sigmoid(_dot(eh, w2_ref[...]) + b2_ref[...])


_gate = pl.pallas_call(
    _gate_body,
    grid=(E // BE,),
    in_specs=[
        pl.BlockSpec((BE, DE), lambda i: (i, 0)),
        pl.BlockSpec((DE, D), lambda i: (0, 0)),
        pl.BlockSpec((1, D), lambda i: (0, 0)),
        pl.BlockSpec((D, 1), lambda i: (0, 0)),
        pl.BlockSpec((1, 1), lambda i: (0, 0)),
    ],
    out_specs=pl.BlockSpec((BE, 1), lambda i: (i, 0)),
    out_shape=jax.ShapeDtypeStruct((E, 1), jnp.float32),
)


def _table_body(x_ref, w_ref, b_ref, o_ref):
    o_ref[...] = _silu(_dot(x_ref[...], w_ref[...]) + b_ref[...])


_table = pl.pallas_call(
    _table_body,
    grid=(N // BN,),
    in_specs=[
        pl.BlockSpec((BN, D), lambda i: (i, 0)),
        pl.BlockSpec((D, D), lambda i: (0, 0)),
        pl.BlockSpec((1, D), lambda i: (0, 0)),
    ],
    out_specs=pl.BlockSpec((BN, D), lambda i: (i, 0)),
    out_shape=jax.ShapeDtypeStruct((N, D), jnp.float32),
)


def _sc_body(y_hbm, src_hbm, dst_hbm, ew_hbm, out_hbm, cnt_hbm,
             src_v, dst_v, ew_v, ones_v, zc_v, rows_v, acc_sh, cnt_sh):
    cid = lax.axis_index("c")
    sid = lax.axis_index("s")
    wid = cid * NS + sid
    zero = jnp.zeros((L,), jnp.float32)
    one = jnp.ones((L,), jnp.float32)

    def _init(i, carry):
        for j in range(D // L):
            rows_v[i, pl.ds(j * L, L)] = zero
        return carry

    lax.fori_loop(0, C, _init, 0)
    for i in range(C // L):
        ones_v[pl.ds(i * L, L)] = one
        zc_v[pl.ds(i * L, L)] = zero

    nrch = N // C
    nown = (nrch + NS - 1) // NS

    def _over_owned(fn):
        def body(k, carry):
            j = sid + NS * k

            @pl.when(j < nrch)
            def _():
                fn(j * C)

            return carry

        lax.fori_loop(0, nown, body, 0)

    def _zero_at(r):
        pltpu.sync_copy(rows_v, acc_sh.at[pl.ds(r, C)])
        pltpu.sync_copy(zc_v, cnt_sh.at[pl.ds(r, C)])

    _over_owned(_zero_at)
    plsc.subcore_barrier()

    ebase = wid * EPW

    def _chunk(k, carry):
        base = ebase + k * C
        pltpu.sync_copy(src_hbm.at[pl.ds(base, C)], src_v)
        pltpu.sync_copy(dst_hbm.at[pl.ds(base, C)], dst_v)
        pltpu.sync_copy(ew_hbm.at[pl.ds(base, C)], ew_v)
        pltpu.sync_copy(y_hbm.at[src_v], rows_v)

        def _scale(g, cc):
            ewv = ew_v[pl.ds(g * L, L)]
            for r in range(L):
                w = jnp.take_along_axis(
                    ewv, jnp.full((L,), r, jnp.int32), axis=0,
                    mode="promise_in_bounds")
                i = g * L + r
                for j in range(D // L):
                    sl = pl.ds(j * L, L)
                    rows_v[i, sl] = rows_v[i, sl] * w
            return cc

        lax.fori_loop(0, C // L, _scale, 0)

        pltpu.sync_copy(rows_v, acc_sh.at[dst_v], add=True)
        pltpu.sync_copy(ones_v, cnt_sh.at[dst_v], add=True)
        return carry

    lax.fori_loop(0, NCHUNK, _chunk, 0)
    plsc.subcore_barrier()

    def _dump_at(r):
        pltpu.sync_copy(acc_sh.at[pl.ds(r, C)], rows_v)
        pltpu.sync_copy(rows_v, out_hbm.at[cid, pl.ds(r, C)])
        pltpu.sync_copy(cnt_sh.at[pl.ds(r, C)], zc_v)
        pltpu.sync_copy(zc_v, cnt_hbm.at[pl.ds(cid * N + r, C)])

    _over_owned(_dump_at)


@functools.cache
def _build_sc_scatter():
    return functools.partial(
        pl.kernel,
        out_type=[
            jax.ShapeDtypeStruct((NC, N, D), jnp.float32),
            jax.ShapeDtypeStruct((NC * N,), jnp.float32),
        ],
        mesh=plsc.VectorSubcoreMesh(
            core_axis_name="c", subcore_axis_name="s",
            num_cores=NC, num_subcores=NS),
        scratch_types=[
            pltpu.VMEM((C,), jnp.int32),
            pltpu.VMEM((C,), jnp.int32),
            pltpu.VMEM((C,), jnp.float32),
            pltpu.VMEM((C,), jnp.float32),
            pltpu.VMEM((C,), jnp.float32),
            pltpu.VMEM((C, D), jnp.float32),
            pltpu.VMEM_SHARED((N, D), jnp.float32),
            pltpu.VMEM_SHARED((N,), jnp.float32),
        ],
    )(_sc_body)


def _final_body(x_ref, p_ref, c_ref, w1x_ref, w1o_ref, b1_ref, w2_ref, b2_ref,
                rw_ref, g_ref, be_ref, o_ref):
    x = x_ref[...]
    p = p_ref[0] + p_ref[1]
    cs = c_ref[0] + c_ref[1]
    cnt = jnp.maximum(cs, 1.0)
    agg = p / cnt
    h1 = _silu(_dot(x, w1x_ref[...]) + _dot(agg, w1o_ref[...]) + b1_ref[...])
    h = _dot(h1, w2_ref[...]) + b2_ref[...] + _dot(x, rw_ref[...])
    mu = jnp.mean(h, axis=-1, keepdims=True)
    var = jnp.mean((h - mu) ** 2, axis=-1, keepdims=True)
    o_ref[...] = (h - mu) * lax.rsqrt(var + 1e-5) * g_ref[...] + be_ref[...]


_final = pl.pallas_call(
    _final_body,
    grid=(N // BN,),
    in_specs=[
        pl.BlockSpec((BN, D), lambda i: (i, 0)),
        pl.BlockSpec((NC, BN, D), lambda i: (0, i, 0)),
        pl.BlockSpec((NC, BN, 1), lambda i: (0, i, 0)),
        pl.BlockSpec((D, D), lambda i: (0, 0)),
        pl.BlockSpec((D, D), lambda i: (0, 0)),
        pl.BlockSpec((1, D), lambda i: (0, 0)),
        pl.BlockSpec((D, D), lambda i: (0, 0)),
        pl.BlockSpec((1, D), lambda i: (0, 0)),
        pl.BlockSpec((D, D), lambda i: (0, 0)),
        pl.BlockSpec((1, D), lambda i: (0, 0)),
        pl.BlockSpec((1, D), lambda i: (0, 0)),
    ],
    out_specs=pl.BlockSpec((BN, D), lambda i: (i, 0)),
    out_shape=jax.ShapeDtypeStruct((N, D), jnp.float32),
)


def kernel(x, edge_index, edge_attr, agg_W, agg_b, e_W1, e_b1, e_W2, e_b2,
           u_W1, u_b1, u_W2, u_b2, gamma, beta, res_W):
    src = jnp.asarray(edge_index[0], jnp.int32)
    dst = jnp.asarray(edge_index[1], jnp.int32)
    ew = _gate(edge_attr, e_W1, e_b1.reshape(1, D), e_W2, e_b2.reshape(1, 1))
    y = _table(x, agg_W, agg_b.reshape(1, D))
    out_p, cnt_p = _build_sc_scatter()(y, src, dst, ew.reshape(E))
    cnt_p = cnt_p.reshape(NC, N, 1)
    return _final(x, out_p, cnt_p, u_W1[:D], u_W1[D:], u_b1.reshape(1, D),
                  u_W2, u_b2.reshape(1, D), res_W, gamma.reshape(1, D),
                  beta.reshape(1, D))
